# baseline (speedup 1.0000x reference)
import functools

import jax
import jax.numpy as jnp
from jax.experimental import pallas as pl
from jax.experimental.pallas import tpu as pltpu


def _gelu(x):
    return 0.5 * x * (1.0 + jax.lax.erf(x * (2.0 ** -0.5)))


def _routing_kernel(x_ref, wg_ref, i1_ref, i2_ref, g1_ref, g2_ref):
    logits = jax.lax.dot_general(x_ref[...], wg_ref[...],
                                 (((1,), (1,)), ((), ())),
                                 preferred_element_type=jnp.float32)
    neg = jnp.finfo(jnp.float32).min
    m1 = jnp.max(logits, axis=1)
    i1 = jnp.argmax(logits, axis=1)
    e_ids = jax.lax.broadcasted_iota(jnp.int32, logits.shape, 1)
    masked = jnp.where(e_ids == i1[:, None], neg, logits)
    m2 = jnp.max(masked, axis=1)
    i2 = jnp.argmax(masked, axis=1)
    g1 = jax.nn.sigmoid(m1 - m2)
    i1_ref[...] = i1[None, :]
    i2_ref[...] = i2[None, :]
    g1_ref[...] = g1[None, :]
    g2_ref[...] = (1.0 - g1)[None, :]


def _moe_ffn_kernel(be_ref, bv_ref, x_ref, tok_ref, gate_ref,
                    w1_ref, b1_ref, w2_ref, b2_ref, out_ref,
                    xg_ref, pacc_ref, *, blk, nf):
    b = pl.program_id(0)
    f = pl.program_id(1)

    @pl.when((b == 0) & (f == 0))
    def _init():
        out_ref[...] = jnp.zeros_like(out_ref)

    @pl.when(bv_ref[b] != 0)
    def _body():
        T = x_ref.shape[0]
        tok = tok_ref[0, 0]
        t_ids = jax.lax.broadcasted_iota(jnp.int32, (blk, T), 1)

        @pl.when(f == 0)
        def _gather():
            onehot = jnp.where(t_ids == tok[:, None], 1.0, 0.0).astype(jnp.bfloat16)
            xg_ref[...] = jax.lax.dot_general(
                onehot, x_ref[...].astype(jnp.bfloat16),
                (((1,), (0,)), ((), ())),
                preferred_element_type=jnp.float32).astype(jnp.bfloat16)
            pacc_ref[...] = jnp.zeros_like(pacc_ref)

        h = jax.lax.dot_general(xg_ref[...], w1_ref[0],
                                (((1,), (1,)), ((), ())),
                                preferred_element_type=jnp.float32)
        h = _gelu(h + b1_ref[0, 0]).astype(jnp.bfloat16)
        pacc_ref[...] += jax.lax.dot_general(
            h, w2_ref[0], (((1,), (1,)), ((), ())),
            preferred_element_type=jnp.float32)

        @pl.when(f == nf - 1)
        def _scatter():
            g = gate_ref[0, 0]
            y = (pacc_ref[...] + b2_ref[0, 0]).astype(jnp.bfloat16)
            scaled = jnp.where(
                t_ids == tok[:, None], g[:, None], 0.0).astype(jnp.bfloat16)
            out_ref[...] += jax.lax.dot_general(
                scaled, y, (((0,), (0,)), ((), ())),
                preferred_element_type=jnp.float32)


def kernel(x, Wg, W1, b1, W2, b2):
    B, T, D = x.shape
    E, F, _ = W1.shape
    K = 2
    x_flat = x.reshape(B * T, D)

    i1, i2, g1, g2 = pl.pallas_call(
        _routing_kernel,
        out_shape=(
            jax.ShapeDtypeStruct((1, T), jnp.int32),
            jax.ShapeDtypeStruct((1, T), jnp.int32),
            jax.ShapeDtypeStruct((1, T), jnp.float32),
            jax.ShapeDtypeStruct((1, T), jnp.float32),
        ),
    )(x_flat, Wg)

    S = K * T
    blk = min(512, T)
    nb = S // blk + E
    ef = jnp.stack([i1[0], i2[0]], axis=1).reshape(S)
    gf = jnp.stack([g1[0], g2[0]], axis=1).reshape(S)
    order = jnp.argsort(ef)
    ef_s = ef[order]
    counts = jnp.bincount(ef, length=E)
    nblk_e = (counts + blk - 1) // blk
    pad_off = (jnp.cumsum(nblk_e) - nblk_e) * blk
    sort_off = jnp.cumsum(counts) - counts
    rank = jnp.arange(S, dtype=jnp.int32) - sort_off[ef_s]
    dest = pad_off[ef_s] + rank
    slot_tok = jnp.zeros((nb * blk,), jnp.int32).at[dest].set(
        (order // K).astype(jnp.int32))
    slot_gate = jnp.zeros((nb * blk,), jnp.float32).at[dest].set(gf[order])
    blocks_cum = jnp.cumsum(nblk_e)
    b_ids = jnp.arange(nb, dtype=jnp.int32)
    block_expert = jnp.clip(
        jnp.searchsorted(blocks_cum, b_ids, side="right"), 0, E - 1
    ).astype(jnp.int32)
    block_valid = (b_ids < blocks_cum[-1]).astype(jnp.int32)

    fb = min(1024, F)
    nf = F // fb
    b1r = b1.reshape(E, nf, 1, fb)
    b2r = b2.reshape(E, 1, D)
    tok_r = slot_tok.reshape(nb, 1, blk)
    gate_r = slot_gate.reshape(nb, 1, blk)

    grid_spec = pltpu.PrefetchScalarGridSpec(
        num_scalar_prefetch=2,
        grid=(nb, nf),
        in_specs=[
            pl.BlockSpec((T, D), lambda b, f, be, bv: (0, 0)),
            pl.BlockSpec((1, 1, blk), lambda b, f, be, bv: (b, 0, 0)),
            pl.BlockSpec((1, 1, blk), lambda b, f, be, bv: (b, 0, 0)),
            pl.BlockSpec((1, fb, D), lambda b, f, be, bv: (be[b], f, 0)),
            pl.BlockSpec((1, 1, 1, fb), lambda b, f, be, bv: (be[b], f, 0, 0)),
            pl.BlockSpec((1, D, fb), lambda b, f, be, bv: (be[b], 0, f)),
            pl.BlockSpec((1, 1, D), lambda b, f, be, bv: (be[b], 0, 0)),
        ],
        out_specs=pl.BlockSpec((T, D), lambda b, f, be, bv: (0, 0)),
        scratch_shapes=[
            pltpu.VMEM((blk, D), jnp.bfloat16),
            pltpu.VMEM((blk, D), jnp.float32),
        ],
    )
    out = pl.pallas_call(
        functools.partial(_moe_ffn_kernel, blk=blk, nf=nf),
        grid_spec=grid_spec,
        out_shape=jax.ShapeDtypeStruct((T, D), jnp.float32),
    )(block_expert, block_valid, x_flat, tok_r, gate_r,
      W1.astype(jnp.bfloat16), b1r, W2.astype(jnp.bfloat16), b2r)
    return out.reshape(B, T, D)


# speedup vs baseline: 1.3748x; 1.3748x over previous
import functools

import jax
import jax.numpy as jnp
from jax.experimental import pallas as pl
from jax.experimental.pallas import tpu as pltpu


def _gelu(x):
    return 0.5 * x * (1.0 + jax.lax.erf(x * (2.0 ** -0.5)))


def _routing_kernel(x_ref, wg_ref, i1_ref, i2_ref, g1_ref, g2_ref):
    logits = jax.lax.dot_general(x_ref[...], wg_ref[...],
                                 (((1,), (1,)), ((), ())),
                                 preferred_element_type=jnp.float32)
    neg = jnp.finfo(jnp.float32).min
    m1 = jnp.max(logits, axis=1)
    i1 = jnp.argmax(logits, axis=1)
    e_ids = jax.lax.broadcasted_iota(jnp.int32, logits.shape, 1)
    masked = jnp.where(e_ids == i1[:, None], neg, logits)
    m2 = jnp.max(masked, axis=1)
    i2 = jnp.argmax(masked, axis=1)
    g1 = jax.nn.sigmoid(m1 - m2)
    i1_ref[...] = i1[None, :]
    i2_ref[...] = i2[None, :]
    g1_ref[...] = g1[None, :]
    g2_ref[...] = (1.0 - g1)[None, :]


def _phys(b, half):
    return jnp.where(b < half, 2 * b, 2 * (b - half) + 1)


def _moe_ffn_kernel(be_ref, bv_ref, x_ref, tok_ref, gate_ref,
                    w1_ref, b1_ref, w2_ref, b2_ref, out_ref,
                    xg_ref, pacc_ref, *, blk, nf, half):
    bg = pl.program_id(0)
    f = pl.program_id(1)
    b = _phys(bg, half)

    @pl.when(((bg == 0) | (bg == half)) & (f == 0))
    def _init():
        out_ref[...] = jnp.zeros_like(out_ref)

    @pl.when(bv_ref[b] != 0)
    def _body():
        T = x_ref.shape[0]
        tok = tok_ref[0, 0]
        t_ids = jax.lax.broadcasted_iota(jnp.int32, (blk, T), 1)

        @pl.when(f == 0)
        def _gather():
            onehot = jnp.where(t_ids == tok[:, None], 1.0, 0.0)
            xg_ref[...] = jax.lax.dot_general(
                onehot, x_ref[...], (((1,), (0,)), ((), ())),
                preferred_element_type=jnp.float32)
            pacc_ref[...] = jnp.zeros_like(pacc_ref)

        h = jax.lax.dot_general(xg_ref[...], w1_ref[0],
                                (((1,), (1,)), ((), ())),
                                preferred_element_type=jnp.float32)
        h = _gelu(h + b1_ref[0, 0])
        pacc_ref[...] += jax.lax.dot_general(
            h, w2_ref[0], (((1,), (1,)), ((), ())),
            preferred_element_type=jnp.float32)

        @pl.when(f == nf - 1)
        def _scatter():
            g = gate_ref[0, 0]
            y = pacc_ref[...] + b2_ref[0, 0]
            scaled = jnp.where(t_ids == tok[:, None], g[:, None], 0.0)
            out_ref[0] += jax.lax.dot_general(
                scaled, y, (((0,), (0,)), ((), ())),
                preferred_element_type=jnp.float32)


def kernel(x, Wg, W1, b1, W2, b2):
    B, T, D = x.shape
    E, F, _ = W1.shape
    K = 2
    x_flat = x.reshape(B * T, D)

    i1, i2, g1, g2 = pl.pallas_call(
        _routing_kernel,
        out_shape=(
            jax.ShapeDtypeStruct((1, T), jnp.int32),
            jax.ShapeDtypeStruct((1, T), jnp.int32),
            jax.ShapeDtypeStruct((1, T), jnp.float32),
            jax.ShapeDtypeStruct((1, T), jnp.float32),
        ),
    )(x_flat, Wg)

    S = K * T
    blk = min(512, T)
    nb = S // blk + E
    ef = jnp.stack([i1[0], i2[0]], axis=1).reshape(S)
    gf = jnp.stack([g1[0], g2[0]], axis=1).reshape(S)
    order = jnp.argsort(ef)
    ef_s = ef[order]
    counts = jnp.bincount(ef, length=E)
    nblk_e = (counts + blk - 1) // blk
    pad_off = (jnp.cumsum(nblk_e) - nblk_e) * blk
    sort_off = jnp.cumsum(counts) - counts
    rank = jnp.arange(S, dtype=jnp.int32) - sort_off[ef_s]
    dest = pad_off[ef_s] + rank
    slot_tok = jnp.zeros((nb * blk,), jnp.int32).at[dest].set(
        (order // K).astype(jnp.int32))
    slot_gate = jnp.zeros((nb * blk,), jnp.float32).at[dest].set(gf[order])
    blocks_cum = jnp.cumsum(nblk_e)
    b_ids = jnp.arange(nb, dtype=jnp.int32)
    block_expert = jnp.clip(
        jnp.searchsorted(blocks_cum, b_ids, side="right"), 0, E - 1
    ).astype(jnp.int32)
    block_valid = (b_ids < blocks_cum[-1]).astype(jnp.int32)

    fb = min(1024, F)
    nf = F // fb
    b1r = b1.reshape(E, nf, 1, fb)
    b2r = b2.reshape(E, 1, D)
    tok_r = slot_tok.reshape(nb, 1, blk)
    gate_r = slot_gate.reshape(nb, 1, blk)

    half = nb // 2
    grid_spec = pltpu.PrefetchScalarGridSpec(
        num_scalar_prefetch=2,
        grid=(nb, nf),
        in_specs=[
            pl.BlockSpec((T, D), lambda b, f, be, bv: (0, 0)),
            pl.BlockSpec((1, 1, blk),
                         lambda b, f, be, bv: (_phys(b, half), 0, 0)),
            pl.BlockSpec((1, 1, blk),
                         lambda b, f, be, bv: (_phys(b, half), 0, 0)),
            pl.BlockSpec((1, fb, D),
                         lambda b, f, be, bv: (be[_phys(b, half)], f, 0)),
            pl.BlockSpec((1, 1, 1, fb),
                         lambda b, f, be, bv: (be[_phys(b, half)], f, 0, 0)),
            pl.BlockSpec((1, D, fb),
                         lambda b, f, be, bv: (be[_phys(b, half)], 0, f)),
            pl.BlockSpec((1, 1, D),
                         lambda b, f, be, bv: (be[_phys(b, half)], 0, 0)),
        ],
        out_specs=pl.BlockSpec(
            (1, T, D), lambda b, f, be, bv: (jnp.where(b < half, 0, 1), 0, 0)),
        scratch_shapes=[
            pltpu.VMEM((blk, D), jnp.float32),
            pltpu.VMEM((blk, D), jnp.float32),
        ],
    )
    out = pl.pallas_call(
        functools.partial(_moe_ffn_kernel, blk=blk, nf=nf, half=half),
        grid_spec=grid_spec,
        out_shape=jax.ShapeDtypeStruct((2, T, D), jnp.float32),
        compiler_params=pltpu.CompilerParams(
            dimension_semantics=("parallel", "arbitrary")),
    )(block_expert, block_valid, x_flat, tok_r, gate_r, W1, b1r, W2, b2r)
    return (out[0] + out[1]).reshape(B, T, D)


# speedup vs baseline: 1.4658x; 1.0662x over previous
import functools

import jax
import jax.numpy as jnp
from jax.experimental import pallas as pl
from jax.experimental.pallas import tpu as pltpu


def _gelu(x):
    return 0.5 * x * (1.0 + jax.lax.erf(x * (2.0 ** -0.5)))


def _routing_kernel(x_ref, wg_ref, i1_ref, i2_ref, g1_ref, g2_ref,
                    r0_ref, r1_ref, c1_ref, c2_ref):
    logits = jax.lax.dot_general(x_ref[...], wg_ref[...],
                                 (((1,), (1,)), ((), ())),
                                 preferred_element_type=jnp.float32)
    neg = jnp.finfo(jnp.float32).min
    m1 = jnp.max(logits, axis=1)
    i1 = jnp.argmax(logits, axis=1)
    e_ids = jax.lax.broadcasted_iota(jnp.int32, logits.shape, 1)
    masked = jnp.where(e_ids == i1[:, None], neg, logits)
    m2 = jnp.max(masked, axis=1)
    i2 = jnp.argmax(masked, axis=1)
    g1 = jax.nn.sigmoid(m1 - m2)
    i1_ref[...] = i1[None, :]
    i2_ref[...] = i2[None, :]
    g1_ref[...] = g1[None, :]
    g2_ref[...] = (1.0 - g1)[None, :]
    oh1 = jnp.where(e_ids == i1[:, None], 1.0, 0.0)
    oh2 = jnp.where(e_ids == i2[:, None], 1.0, 0.0)
    T, E = oh1.shape
    tri = jnp.where(
        jax.lax.broadcasted_iota(jnp.int32, (T, T), 0)
        > jax.lax.broadcasted_iota(jnp.int32, (T, T), 1),
        1.0, 0.0).astype(jnp.bfloat16)
    ohb = jnp.concatenate([oh1, oh2], axis=1).astype(jnp.bfloat16)
    csum = jax.lax.dot_general(tri, ohb, (((1,), (0,)), ((), ())),
                               preferred_element_type=jnp.float32)
    c1x = csum[:, :E]
    c2x = csum[:, E:]
    r0_ref[...] = jnp.sum(oh1 * c1x, axis=1).astype(jnp.int32)[None, :]
    r1_ref[...] = jnp.sum(oh2 * c2x, axis=1).astype(jnp.int32)[None, :]
    c1_ref[...] = jnp.sum(oh1, axis=0).astype(jnp.int32)[None, :]
    c2_ref[...] = jnp.sum(oh2, axis=0).astype(jnp.int32)[None, :]


def _moe_ffn_kernel(be_ref, bv_ref, x_ref, tok_ref, gate_ref,
                    w1_ref, b1_ref, w2_ref, b2_ref, out_ref,
                    xg_ref, pacc_ref, *, blk, nf):
    b = pl.program_id(0)
    f = pl.program_id(1)

    @pl.when((b == 0) & (f == 0))
    def _init():
        out_ref[...] = jnp.zeros_like(out_ref)

    @pl.when(bv_ref[b] != 0)
    def _body():
        T = x_ref.shape[0]
        tok = tok_ref[0, 0]
        t_ids = jax.lax.broadcasted_iota(jnp.int32, (blk, T), 1)

        @pl.when(f == 0)
        def _gather():
            onehot = jnp.where(t_ids == tok[:, None], 1.0, 0.0)
            xg_ref[...] = jax.lax.dot_general(
                onehot, x_ref[...], (((1,), (0,)), ((), ())),
                preferred_element_type=jnp.float32)
            pacc_ref[...] = jnp.zeros_like(pacc_ref)

        h = jax.lax.dot_general(xg_ref[...], w1_ref[0],
                                (((1,), (1,)), ((), ())),
                                preferred_element_type=jnp.float32)
        h = _gelu(h + b1_ref[0, 0])
        pacc_ref[...] += jax.lax.dot_general(
            h, w2_ref[0], (((1,), (1,)), ((), ())),
            preferred_element_type=jnp.float32)

        @pl.when(f == nf - 1)
        def _scatter():
            g = gate_ref[0, 0]
            y = pacc_ref[...] + b2_ref[0, 0]
            scaled = jnp.where(t_ids == tok[:, None], g[:, None], 0.0)
            out_ref[...] += jax.lax.dot_general(
                scaled, y, (((0,), (0,)), ((), ())),
                preferred_element_type=jnp.float32)


def kernel(x, Wg, W1, b1, W2, b2):
    B, T, D = x.shape
    E, F, _ = W1.shape
    K = 2
    x_flat = x.reshape(B * T, D)

    i1, i2, g1, g2, r0, r1, c1, c2 = pl.pallas_call(
        _routing_kernel,
        out_shape=(
            jax.ShapeDtypeStruct((1, T), jnp.int32),
            jax.ShapeDtypeStruct((1, T), jnp.int32),
            jax.ShapeDtypeStruct((1, T), jnp.float32),
            jax.ShapeDtypeStruct((1, T), jnp.float32),
            jax.ShapeDtypeStruct((1, T), jnp.int32),
            jax.ShapeDtypeStruct((1, T), jnp.int32),
            jax.ShapeDtypeStruct((1, E), jnp.int32),
            jax.ShapeDtypeStruct((1, E), jnp.int32),
        ),
    )(x_flat, Wg)

    S = K * T
    blk = min(512, T)
    nb = S // blk + E
    counts = (c1[0] + c2[0]).astype(jnp.int32)
    nblk_e = (counts + blk - 1) // blk
    pad_off = (jnp.cumsum(nblk_e) - nblk_e) * blk
    dest0 = pad_off[i1[0]] + r0[0]
    dest1 = pad_off[i2[0]] + c1[0][i2[0]] + r1[0]
    dest = jnp.concatenate([dest0, dest1])
    tok_ids = jnp.arange(T, dtype=jnp.int32)
    slot_tok = jnp.zeros((nb * blk,), jnp.int32).at[dest].set(
        jnp.concatenate([tok_ids, tok_ids]))
    slot_gate = jnp.zeros((nb * blk,), jnp.float32).at[dest].set(
        jnp.concatenate([g1[0], g2[0]]))
    blocks_cum = jnp.cumsum(nblk_e)
    b_ids = jnp.arange(nb, dtype=jnp.int32)
    block_expert = jnp.clip(
        jnp.searchsorted(blocks_cum, b_ids, side="right"), 0, E - 1
    ).astype(jnp.int32)
    block_valid = (b_ids < blocks_cum[-1]).astype(jnp.int32)

    fb = min(1024, F)
    nf = F // fb
    b1r = b1.reshape(E, nf, 1, fb)
    b2r = b2.reshape(E, 1, D)
    tok_r = slot_tok.reshape(nb, 1, blk)
    gate_r = slot_gate.reshape(nb, 1, blk)

    grid_spec = pltpu.PrefetchScalarGridSpec(
        num_scalar_prefetch=2,
        grid=(nb, nf),
        in_specs=[
            pl.BlockSpec((T, D), lambda b, f, be, bv: (0, 0)),
            pl.BlockSpec((1, 1, blk), lambda b, f, be, bv: (b, 0, 0)),
            pl.BlockSpec((1, 1, blk), lambda b, f, be, bv: (b, 0, 0)),
            pl.BlockSpec((1, fb, D), lambda b, f, be, bv: (be[b], f, 0)),
            pl.BlockSpec((1, 1, 1, fb), lambda b, f, be, bv: (be[b], f, 0, 0)),
            pl.BlockSpec((1, D, fb), lambda b, f, be, bv: (be[b], 0, f)),
            pl.BlockSpec((1, 1, D), lambda b, f, be, bv: (be[b], 0, 0)),
        ],
        out_specs=pl.BlockSpec((T, D), lambda b, f, be, bv: (0, 0)),
        scratch_shapes=[
            pltpu.VMEM((blk, D), jnp.float32),
            pltpu.VMEM((blk, D), jnp.float32),
        ],
    )
    out = pl.pallas_call(
        functools.partial(_moe_ffn_kernel, blk=blk, nf=nf),
        grid_spec=grid_spec,
        out_shape=jax.ShapeDtypeStruct((T, D), jnp.float32),
    )(block_expert, block_valid, x_flat, tok_r, gate_r, W1, b1r, W2, b2r)
    return out.reshape(B, T, D)
